# baseline (speedup 1.0000x reference)
import jax
import jax.numpy as jnp
from jax import lax
from jax.experimental import pallas as pl
from jax.experimental.pallas import tpu as pltpu
from jax.experimental.pallas import tpu_sc as plsc

N_ROWS = 320000
D = 128
NC = 2
NS = 16
NW = NC * NS
LANES = 16
G = D // LANES
ROWS_PER_W = N_ROWS // NW
CHUNK_ROWS = 250
CHUNK_WORDS = CHUNK_ROWS * D
NCHUNK = ROWS_PER_W // CHUNK_ROWS
WORDS_PER_W = ROWS_PER_W * D
U = 5

_MESH = plsc.VectorSubcoreMesh(core_axis_name="c", subcore_axis_name="s")


def _wid():
    return lax.axis_index("c") * NS + lax.axis_index("s")


def _minmax_body(x_hbm, pmin_hbm, pmax_hbm, buf0, buf1, stage, sem0, sem1):
    wid = _wid()
    base = wid * WORDS_PER_W

    def dma(c, buf, sem):
        return pltpu.make_async_copy(
            x_hbm.at[pl.ds(base + c * CHUNK_WORDS, CHUNK_WORDS)], buf, sem)

    dma(0, buf0, sem0).start()
    dma(1, buf1, sem1).start()

    inf = jnp.full((LANES,), jnp.inf, jnp.float32)
    ninf = jnp.full((LANES,), -jnp.inf, jnp.float32)
    acc0 = tuple([inf] * G) + tuple([ninf] * G)

    def chunk_compute(buf, acc):
        def row_body(i, a):
            mins = list(a[:G])
            maxs = list(a[G:])
            for u in range(U):
                roff = (i * U + u) * D
                for g in range(G):
                    v = buf[pl.ds(roff + g * LANES, LANES)]
                    mins[g] = jnp.minimum(mins[g], v)
                    maxs[g] = jnp.maximum(maxs[g], v)
            return tuple(mins) + tuple(maxs)
        return lax.fori_loop(0, CHUNK_ROWS // U, row_body, acc)

    def pair_body(p, acc):
        c = p * 2
        dma(c, buf0, sem0).wait()
        acc = chunk_compute(buf0, acc)

        @pl.when(c + 2 < NCHUNK)
        def _():
            dma(c + 2, buf0, sem0).start()

        dma(c + 1, buf1, sem1).wait()
        acc = chunk_compute(buf1, acc)

        @pl.when(c + 3 < NCHUNK)
        def _():
            dma(c + 3, buf1, sem1).start()

        return acc

    acc = lax.fori_loop(0, NCHUNK // 2, pair_body, acc0)

    for g in range(G):
        stage[pl.ds(g * LANES, LANES)] = acc[g]
    pltpu.sync_copy(stage, pmin_hbm.at[pl.ds(wid * D, D)])
    for g in range(G):
        stage[pl.ds(g * LANES, LANES)] = acc[G + g]
    pltpu.sync_copy(stage, pmax_hbm.at[pl.ds(wid * D, D)])


def _relu_body(x_hbm, pmin_hbm, pmax_hbm, t_hbm, prelu_hbm,
               buf0, buf1, pm_v, px_v, t_v, stage, sem0, sem1):
    wid = _wid()
    base = wid * WORDS_PER_W

    def dma(c, buf, sem):
        return pltpu.make_async_copy(
            x_hbm.at[pl.ds(base + c * CHUNK_WORDS, CHUNK_WORDS)], buf, sem)

    dma(0, buf0, sem0).start()
    dma(1, buf1, sem1).start()

    zero = jnp.zeros((LANES,), jnp.float32)

    pltpu.sync_copy(pmin_hbm, pm_v)
    pltpu.sync_copy(pmax_hbm, px_v)
    pltpu.sync_copy(t_hbm, t_v)

    inf = jnp.full((LANES,), jnp.inf, jnp.float32)
    ninf = jnp.full((LANES,), -jnp.inf, jnp.float32)

    def red_body(w, a):
        mins = list(a[:G])
        maxs = list(a[G:])
        for g in range(G):
            mins[g] = jnp.minimum(mins[g], pm_v[pl.ds(w * D + g * LANES, LANES)])
            maxs[g] = jnp.maximum(maxs[g], px_v[pl.ds(w * D + g * LANES, LANES)])
        return tuple(mins) + tuple(maxs)

    red = lax.fori_loop(0, NW, red_body, tuple([inf] * G) + tuple([ninf] * G))
    bias = []
    for g in range(G):
        tg = t_v[pl.ds(g * LANES, LANES)]
        bias.append(tg * red[G + g] + (1.0 - tg) * red[g])
    bias = tuple(bias)

    acc0 = tuple([zero] * G)

    def chunk_compute(buf, acc):
        def row_body(i, a):
            sums = list(a)
            for u in range(U):
                roff = (i * U + u) * D
                for g in range(G):
                    v = buf[pl.ds(roff + g * LANES, LANES)]
                    sums[g] = sums[g] + jnp.maximum(v - bias[g], 0.0)
            return tuple(sums)
        return lax.fori_loop(0, CHUNK_ROWS // U, row_body, acc)

    def pair_body(p, acc):
        c = p * 2
        dma(c, buf0, sem0).wait()
        acc = chunk_compute(buf0, acc)

        @pl.when(c + 2 < NCHUNK)
        def _():
            dma(c + 2, buf0, sem0).start()

        dma(c + 1, buf1, sem1).wait()
        acc = chunk_compute(buf1, acc)

        @pl.when(c + 3 < NCHUNK)
        def _():
            dma(c + 3, buf1, sem1).start()

        return acc

    acc = lax.fori_loop(0, NCHUNK // 2, pair_body, acc0)

    for g in range(G):
        stage[pl.ds(g * LANES, LANES)] = acc[g]
    pltpu.sync_copy(stage, prelu_hbm.at[pl.ds(wid * D, D)])


def _final_body(pmin_hbm, pmax_hbm, prelu_hbm, t_hbm, sc_hbm, out_hbm,
                pm_v, px_v, pr_v, t_v, sc_v, stage):
    wid = _wid()

    @pl.when(wid == 0)
    def _():
        pltpu.sync_copy(pmin_hbm, pm_v)
        pltpu.sync_copy(pmax_hbm, px_v)
        pltpu.sync_copy(prelu_hbm, pr_v)
        pltpu.sync_copy(t_hbm, t_v)
        pltpu.sync_copy(sc_hbm, sc_v)

        inf = jnp.full((LANES,), jnp.inf, jnp.float32)
        ninf = jnp.full((LANES,), -jnp.inf, jnp.float32)
        zero = jnp.zeros((LANES,), jnp.float32)

        def red_body(w, a):
            mins = list(a[:G])
            maxs = list(a[G:2 * G])
            sums = list(a[2 * G:])
            for g in range(G):
                off = w * D + g * LANES
                mins[g] = jnp.minimum(mins[g], pm_v[pl.ds(off, LANES)])
                maxs[g] = jnp.maximum(maxs[g], px_v[pl.ds(off, LANES)])
                sums[g] = sums[g] + pr_v[pl.ds(off, LANES)]
            return tuple(mins) + tuple(maxs) + tuple(sums)

        red = lax.fori_loop(
            0, NW, red_body,
            tuple([inf] * G) + tuple([ninf] * G) + tuple([zero] * G))

        cnt = jnp.float32(N_ROWS)
        for g in range(G):
            s0 = sc_v[pl.ds(0 * D + g * LANES, LANES)]
            s1 = sc_v[pl.ds(1 * D + g * LANES, LANES)]
            s2 = sc_v[pl.ds(2 * D + g * LANES, LANES)]
            s3 = sc_v[pl.ds(3 * D + g * LANES, LANES)]
            row = (s0 * cnt + s1 * red[g] + s2 * red[G + g]
                   + s3 * red[2 * G + g])
            stage[pl.ds(g * LANES, LANES)] = row
        pltpu.sync_copy(stage, out_hbm)


_minmax_call = pl.kernel(
    _minmax_body,
    out_type=(jax.ShapeDtypeStruct((NW * D,), jnp.float32),
              jax.ShapeDtypeStruct((NW * D,), jnp.float32)),
    mesh=_MESH,
    scratch_types=[
        pltpu.VMEM((CHUNK_WORDS,), jnp.float32),
        pltpu.VMEM((CHUNK_WORDS,), jnp.float32),
        pltpu.VMEM((D,), jnp.float32),
        pltpu.SemaphoreType.DMA,
        pltpu.SemaphoreType.DMA,
    ],
)

_relu_call = pl.kernel(
    _relu_body,
    out_type=jax.ShapeDtypeStruct((NW * D,), jnp.float32),
    mesh=_MESH,
    scratch_types=[
        pltpu.VMEM((CHUNK_WORDS,), jnp.float32),
        pltpu.VMEM((CHUNK_WORDS,), jnp.float32),
        pltpu.VMEM((NW * D,), jnp.float32),
        pltpu.VMEM((NW * D,), jnp.float32),
        pltpu.VMEM((D,), jnp.float32),
        pltpu.VMEM((D,), jnp.float32),
        pltpu.SemaphoreType.DMA,
        pltpu.SemaphoreType.DMA,
    ],
)

_final_call = pl.kernel(
    _final_body,
    out_type=jax.ShapeDtypeStruct((D,), jnp.float32),
    mesh=_MESH,
    scratch_types=[
        pltpu.VMEM((NW * D,), jnp.float32),
        pltpu.VMEM((NW * D,), jnp.float32),
        pltpu.VMEM((NW * D,), jnp.float32),
        pltpu.VMEM((D,), jnp.float32),
        pltpu.VMEM((4 * D,), jnp.float32),
        pltpu.VMEM((D,), jnp.float32),
    ],
)


@jax.jit
def kernel(x, scalars, t):
    x1 = x.reshape(-1)
    pmin, pmax = _minmax_call(x1)
    zfill = jnp.broadcast_to(pmin[0] * 0.0, (N_ROWS + 1, D))
    prelu = _relu_call(x1, pmin, pmax, t)
    row0 = _final_call(pmin, pmax, prelu, t, scalars.reshape(-1))
    return lax.dynamic_update_slice(zfill, row0.reshape(1, D), (0, 0))


# speedup vs baseline: 1.0490x; 1.0490x over previous
import jax
import jax.numpy as jnp
from jax import lax
from jax.experimental import pallas as pl
from jax.experimental.pallas import tpu as pltpu
from jax.experimental.pallas import tpu_sc as plsc

N_ROWS = 320000
D = 128
NC = 2
NS = 16
NW = NC * NS
LANES = 16
G = D // LANES
ROWS_PER_W = N_ROWS // NW
CHUNK_ROWS = 250
CHUNK_WORDS = CHUNK_ROWS * D
NCHUNK = ROWS_PER_W // CHUNK_ROWS
WORDS_PER_W = ROWS_PER_W * D
U = 5

_MESH = plsc.VectorSubcoreMesh(core_axis_name="c", subcore_axis_name="s")


def _wid():
    return lax.axis_index("c") * NS + lax.axis_index("s")


def _minmax_body(x_hbm, pmin_hbm, pmax_hbm, buf0, buf1, stage, sem0, sem1):
    wid = _wid()
    base = wid * WORDS_PER_W

    def dma(c, buf, sem):
        return pltpu.make_async_copy(
            x_hbm.at[pl.ds(base + c * CHUNK_WORDS, CHUNK_WORDS)], buf, sem)

    dma(0, buf0, sem0).start()
    dma(1, buf1, sem1).start()

    inf = jnp.full((LANES,), jnp.inf, jnp.float32)
    ninf = jnp.full((LANES,), -jnp.inf, jnp.float32)
    acc0 = tuple([inf] * G) + tuple([ninf] * G)

    def chunk_compute(buf, acc):
        def row_body(i, a):
            mins = list(a[:G])
            maxs = list(a[G:])
            for u in range(U):
                roff = (i * U + u) * D
                for g in range(G):
                    v = buf[pl.ds(roff + g * LANES, LANES)]
                    mins[g] = jnp.minimum(mins[g], v)
                    maxs[g] = jnp.maximum(maxs[g], v)
            return tuple(mins) + tuple(maxs)
        return lax.fori_loop(0, CHUNK_ROWS // U, row_body, acc)

    def pair_body(p, acc):
        c = p * 2
        dma(c, buf0, sem0).wait()
        acc = chunk_compute(buf0, acc)

        @pl.when(c + 2 < NCHUNK)
        def _():
            dma(c + 2, buf0, sem0).start()

        dma(c + 1, buf1, sem1).wait()
        acc = chunk_compute(buf1, acc)

        @pl.when(c + 3 < NCHUNK)
        def _():
            dma(c + 3, buf1, sem1).start()

        return acc

    acc = lax.fori_loop(0, NCHUNK // 2, pair_body, acc0)

    for g in range(G):
        stage[pl.ds(g * LANES, LANES)] = acc[g]
    pltpu.sync_copy(stage, pmin_hbm.at[pl.ds(wid * D, D)])
    for g in range(G):
        stage[pl.ds(g * LANES, LANES)] = acc[G + g]
    pltpu.sync_copy(stage, pmax_hbm.at[pl.ds(wid * D, D)])


def _relu_body(x_hbm, pmin_hbm, pmax_hbm, t_hbm, sc_hbm,
               big_hbm, out2_hbm,
               buf0, buf1, pm_v, px_v, t_v, stage, zbuf, pr_v, sc_v, shared,
               sem0, sem1, zsem0, zsem1):
    wid = _wid()
    base = wid * WORDS_PER_W
    zbase = (wid * ROWS_PER_W + 1) * D

    def dma(c, buf, sem):
        return pltpu.make_async_copy(
            x_hbm.at[pl.ds(base + c * CHUNK_WORDS, CHUNK_WORDS)], buf, sem)

    def zdma(c, sem):
        return pltpu.make_async_copy(
            zbuf, big_hbm.at[pl.ds(zbase + c * CHUNK_WORDS, CHUNK_WORDS)], sem)

    dma(0, buf0, sem0).start()
    dma(1, buf1, sem1).start()

    zero = jnp.zeros((LANES,), jnp.float32)

    def zinit(i, _):
        for u in range(8):
            zbuf[pl.ds((i * 8 + u) * LANES, LANES)] = zero
        return 0

    lax.fori_loop(0, CHUNK_WORDS // (8 * LANES), zinit, 0)

    pltpu.sync_copy(pmin_hbm, pm_v)
    pltpu.sync_copy(pmax_hbm, px_v)
    pltpu.sync_copy(t_hbm, t_v)

    inf = jnp.full((LANES,), jnp.inf, jnp.float32)
    ninf = jnp.full((LANES,), -jnp.inf, jnp.float32)

    def red_body(w, a):
        mins = list(a[:G])
        maxs = list(a[G:])
        for g in range(G):
            mins[g] = jnp.minimum(mins[g], pm_v[pl.ds(w * D + g * LANES, LANES)])
            maxs[g] = jnp.maximum(maxs[g], px_v[pl.ds(w * D + g * LANES, LANES)])
        return tuple(mins) + tuple(maxs)

    red = lax.fori_loop(0, NW, red_body, tuple([inf] * G) + tuple([ninf] * G))
    bias = []
    for g in range(G):
        tg = t_v[pl.ds(g * LANES, LANES)]
        bias.append(tg * red[G + g] + (1.0 - tg) * red[g])
    bias = tuple(bias)

    acc0 = tuple([zero] * G)

    def chunk_compute(buf, acc):
        def row_body(i, a):
            sums = list(a)
            for u in range(U):
                roff = (i * U + u) * D
                for g in range(G):
                    v = buf[pl.ds(roff + g * LANES, LANES)]
                    sums[g] = sums[g] + jnp.maximum(v - bias[g], 0.0)
            return tuple(sums)
        return lax.fori_loop(0, CHUNK_ROWS // U, row_body, acc)

    def pair_body(p, acc):
        c = p * 2
        dma(c, buf0, sem0).wait()
        acc = chunk_compute(buf0, acc)

        @pl.when(c + 2 < NCHUNK)
        def _():
            dma(c + 2, buf0, sem0).start()

        @pl.when(p > 0)
        def _():
            zdma(c - 2, zsem0).wait()

        zdma(c, zsem0).start()

        dma(c + 1, buf1, sem1).wait()
        acc = chunk_compute(buf1, acc)

        @pl.when(c + 3 < NCHUNK)
        def _():
            dma(c + 3, buf1, sem1).start()

        @pl.when(p > 0)
        def _():
            zdma(c - 1, zsem1).wait()

        zdma(c + 1, zsem1).start()

        return acc

    acc = lax.fori_loop(0, NCHUNK // 2, pair_body, acc0)
    zdma(NCHUNK - 2, zsem0).wait()
    zdma(NCHUNK - 1, zsem1).wait()

    sid = lax.axis_index("s")
    for g in range(G):
        stage[pl.ds(g * LANES, LANES)] = acc[g]
    pltpu.sync_copy(stage, shared.at[pl.ds(sid * D, D)])

    plsc.subcore_barrier()

    cid = lax.axis_index("c")

    @pl.when(sid == 0)
    def _():
        pltpu.sync_copy(shared, pr_v)
        pltpu.sync_copy(sc_hbm, sc_v)

        def sum_body(w, a):
            return tuple(a[g] + pr_v[pl.ds(w * D + g * LANES, LANES)]
                         for g in range(G))

        ssum = lax.fori_loop(0, NS, sum_body, tuple([zero] * G))
        cnt = jnp.float32(N_ROWS)
        on_core0 = (cid == 0).astype(jnp.float32)
        for g in range(G):
            s0 = sc_v[pl.ds(0 * D + g * LANES, LANES)]
            s1 = sc_v[pl.ds(1 * D + g * LANES, LANES)]
            s2 = sc_v[pl.ds(2 * D + g * LANES, LANES)]
            s3 = sc_v[pl.ds(3 * D + g * LANES, LANES)]
            row = (s3 * ssum[g]
                   + on_core0 * (s0 * cnt + s1 * red[g] + s2 * red[G + g]))
            stage[pl.ds(g * LANES, LANES)] = row
        pltpu.sync_copy(stage, out2_hbm.at[pl.ds(cid * D, D)])


_minmax_call = pl.kernel(
    _minmax_body,
    out_type=(jax.ShapeDtypeStruct((NW * D,), jnp.float32),
              jax.ShapeDtypeStruct((NW * D,), jnp.float32)),
    mesh=_MESH,
    scratch_types=[
        pltpu.VMEM((CHUNK_WORDS,), jnp.float32),
        pltpu.VMEM((CHUNK_WORDS,), jnp.float32),
        pltpu.VMEM((D,), jnp.float32),
        pltpu.SemaphoreType.DMA,
        pltpu.SemaphoreType.DMA,
    ],
)

_relu_call = pl.kernel(
    _relu_body,
    out_type=(jax.ShapeDtypeStruct(((N_ROWS + 1) * D,), jnp.float32),
              jax.ShapeDtypeStruct((NC * D,), jnp.float32)),
    mesh=_MESH,
    scratch_types=[
        pltpu.VMEM((CHUNK_WORDS,), jnp.float32),
        pltpu.VMEM((CHUNK_WORDS,), jnp.float32),
        pltpu.VMEM((NW * D,), jnp.float32),
        pltpu.VMEM((NW * D,), jnp.float32),
        pltpu.VMEM((D,), jnp.float32),
        pltpu.VMEM((D,), jnp.float32),
        pltpu.VMEM((CHUNK_WORDS,), jnp.float32),
        pltpu.VMEM((NS * D,), jnp.float32),
        pltpu.VMEM((4 * D,), jnp.float32),
        pltpu.VMEM_SHARED((NS * D,), jnp.float32),
        pltpu.SemaphoreType.DMA,
        pltpu.SemaphoreType.DMA,
        pltpu.SemaphoreType.DMA,
        pltpu.SemaphoreType.DMA,
    ],
)


@jax.jit
def kernel(x, scalars, t):
    x1 = x.reshape(-1)
    pmin, pmax = _minmax_call(x1)
    big, out2 = _relu_call(x1, pmin, pmax, t, scalars.reshape(-1))
    row0 = out2[:D] + out2[D:]
    return lax.dynamic_update_slice(
        big.reshape(N_ROWS + 1, D), row0.reshape(1, D), (0, 0))


# speedup vs baseline: 1.0891x; 1.0382x over previous
import jax
import jax.numpy as jnp
from jax import lax
from jax.experimental import pallas as pl
from jax.experimental.pallas import tpu as pltpu
from jax.experimental.pallas import tpu_sc as plsc

N_ROWS = 320000
D = 128
NC = 2
NS = 16
NW = NC * NS
LANES = 16
G = D // LANES
ROWS_PER_W = N_ROWS // NW
CHUNK_ROWS = 250
CHUNK_WORDS = CHUNK_ROWS * D
NCHUNK = ROWS_PER_W // CHUNK_ROWS
WORDS_PER_W = ROWS_PER_W * D
U = 5

_MESH = plsc.VectorSubcoreMesh(core_axis_name="c", subcore_axis_name="s")


def _wid():
    return lax.axis_index("c") * NS + lax.axis_index("s")


def _minmax_body(x_hbm, pmin_hbm, pmax_hbm, big_hbm,
                 buf0, buf1, stage, zbuf, sem0, sem1, zsem):
    wid = _wid()
    base = wid * WORDS_PER_W
    zbase = (wid * (ROWS_PER_W // 2) + 1) * D

    def dma(c, buf, sem):
        return pltpu.make_async_copy(
            x_hbm.at[pl.ds(base + c * CHUNK_WORDS, CHUNK_WORDS)], buf, sem)

    def zdma(c):
        return pltpu.make_async_copy(
            zbuf, big_hbm.at[pl.ds(zbase + c * CHUNK_WORDS, CHUNK_WORDS)],
            zsem)

    dma(0, buf0, sem0).start()
    dma(1, buf1, sem1).start()

    zero = jnp.zeros((LANES,), jnp.float32)

    def zinit(i, _):
        for u in range(8):
            zbuf[pl.ds((i * 8 + u) * LANES, LANES)] = zero
        return 0

    lax.fori_loop(0, CHUNK_WORDS // (8 * LANES), zinit, 0)

    inf = jnp.full((LANES,), jnp.inf, jnp.float32)
    ninf = jnp.full((LANES,), -jnp.inf, jnp.float32)
    acc0 = tuple([inf] * G) + tuple([ninf] * G)

    def chunk_compute(buf, acc):
        def row_body(i, a):
            mins = list(a[:G])
            maxs = list(a[G:])
            for u in range(U):
                roff = (i * U + u) * D
                for g in range(G):
                    v = buf[pl.ds(roff + g * LANES, LANES)]
                    mins[g] = jnp.minimum(mins[g], v)
                    maxs[g] = jnp.maximum(maxs[g], v)
            return tuple(mins) + tuple(maxs)
        return lax.fori_loop(0, CHUNK_ROWS // U, row_body, acc)

    def pair_body(p, acc):
        c = p * 2
        dma(c, buf0, sem0).wait()
        acc = chunk_compute(buf0, acc)

        @pl.when(c + 2 < NCHUNK)
        def _():
            dma(c + 2, buf0, sem0).start()

        @pl.when(p > 0)
        def _():
            zdma(p - 1).wait()

        zdma(p).start()

        dma(c + 1, buf1, sem1).wait()
        acc = chunk_compute(buf1, acc)

        @pl.when(c + 3 < NCHUNK)
        def _():
            dma(c + 3, buf1, sem1).start()

        return acc

    acc = lax.fori_loop(0, NCHUNK // 2, pair_body, acc0)
    zdma(NCHUNK // 2 - 1).wait()

    for g in range(G):
        stage[pl.ds(g * LANES, LANES)] = acc[g]
    pltpu.sync_copy(stage, pmin_hbm.at[pl.ds(wid * D, D)])
    for g in range(G):
        stage[pl.ds(g * LANES, LANES)] = acc[G + g]
    pltpu.sync_copy(stage, pmax_hbm.at[pl.ds(wid * D, D)])


def _relu_body(x_hbm, pmin_hbm, pmax_hbm, t_hbm, sc_hbm, big_hbm,
               out2_hbm,
               buf0, buf1, pm_v, px_v, t_v, stage, zbuf, pr_v, sc_v, shared,
               sem0, sem1, zsem):
    wid = _wid()
    base = wid * WORDS_PER_W
    zbase = ((N_ROWS // 2) + wid * (ROWS_PER_W // 2) + 1) * D

    def dma(c, buf, sem):
        return pltpu.make_async_copy(
            x_hbm.at[pl.ds(base + c * CHUNK_WORDS, CHUNK_WORDS)], buf, sem)

    def zdma(c):
        return pltpu.make_async_copy(
            zbuf, big_hbm.at[pl.ds(zbase + c * CHUNK_WORDS, CHUNK_WORDS)],
            zsem)

    dma(0, buf0, sem0).start()
    dma(1, buf1, sem1).start()

    zero = jnp.zeros((LANES,), jnp.float32)

    def zinit(i, _):
        for u in range(8):
            zbuf[pl.ds((i * 8 + u) * LANES, LANES)] = zero
        return 0

    lax.fori_loop(0, CHUNK_WORDS // (8 * LANES), zinit, 0)

    pltpu.sync_copy(pmin_hbm, pm_v)
    pltpu.sync_copy(pmax_hbm, px_v)
    pltpu.sync_copy(t_hbm, t_v)

    inf = jnp.full((LANES,), jnp.inf, jnp.float32)
    ninf = jnp.full((LANES,), -jnp.inf, jnp.float32)

    def red_body(w, a):
        mins = list(a[:G])
        maxs = list(a[G:])
        for g in range(G):
            mins[g] = jnp.minimum(mins[g], pm_v[pl.ds(w * D + g * LANES, LANES)])
            maxs[g] = jnp.maximum(maxs[g], px_v[pl.ds(w * D + g * LANES, LANES)])
        return tuple(mins) + tuple(maxs)

    red = lax.fori_loop(0, NW, red_body, tuple([inf] * G) + tuple([ninf] * G))
    bias = []
    for g in range(G):
        tg = t_v[pl.ds(g * LANES, LANES)]
        bias.append(tg * red[G + g] + (1.0 - tg) * red[g])
    bias = tuple(bias)

    acc0 = tuple([zero] * G)

    def chunk_compute(buf, acc):
        def row_body(i, a):
            sums = list(a)
            for u in range(U):
                roff = (i * U + u) * D
                for g in range(G):
                    v = buf[pl.ds(roff + g * LANES, LANES)]
                    sums[g] = sums[g] + jnp.maximum(v - bias[g], 0.0)
            return tuple(sums)
        return lax.fori_loop(0, CHUNK_ROWS // U, row_body, acc)

    def pair_body(p, acc):
        c = p * 2
        dma(c, buf0, sem0).wait()
        acc = chunk_compute(buf0, acc)

        @pl.when(c + 2 < NCHUNK)
        def _():
            dma(c + 2, buf0, sem0).start()

        @pl.when(p > 0)
        def _():
            zdma(p - 1).wait()

        zdma(p).start()

        dma(c + 1, buf1, sem1).wait()
        acc = chunk_compute(buf1, acc)

        @pl.when(c + 3 < NCHUNK)
        def _():
            dma(c + 3, buf1, sem1).start()

        return acc

    acc = lax.fori_loop(0, NCHUNK // 2, pair_body, acc0)
    zdma(NCHUNK // 2 - 1).wait()

    sid = lax.axis_index("s")
    for g in range(G):
        stage[pl.ds(g * LANES, LANES)] = acc[g]
    pltpu.sync_copy(stage, shared.at[pl.ds(sid * D, D)])

    plsc.subcore_barrier()

    cid = lax.axis_index("c")

    @pl.when(sid == 0)
    def _():
        pltpu.sync_copy(shared, pr_v)
        pltpu.sync_copy(sc_hbm, sc_v)

        def sum_body(w, a):
            return tuple(a[g] + pr_v[pl.ds(w * D + g * LANES, LANES)]
                         for g in range(G))

        ssum = lax.fori_loop(0, NS, sum_body, tuple([zero] * G))
        cnt = jnp.float32(N_ROWS)
        on_core0 = (cid == 0).astype(jnp.float32)
        for g in range(G):
            s0 = sc_v[pl.ds(0 * D + g * LANES, LANES)]
            s1 = sc_v[pl.ds(1 * D + g * LANES, LANES)]
            s2 = sc_v[pl.ds(2 * D + g * LANES, LANES)]
            s3 = sc_v[pl.ds(3 * D + g * LANES, LANES)]
            row = (s3 * ssum[g]
                   + on_core0 * (s0 * cnt + s1 * red[g] + s2 * red[G + g]))
            stage[pl.ds(g * LANES, LANES)] = row
        pltpu.sync_copy(stage, out2_hbm.at[pl.ds(cid * D, D)])


_minmax_call = pl.kernel(
    _minmax_body,
    out_type=(jax.ShapeDtypeStruct((NW * D,), jnp.float32),
              jax.ShapeDtypeStruct((NW * D,), jnp.float32),
              jax.ShapeDtypeStruct(((N_ROWS + 1) * D,), jnp.float32)),
    mesh=_MESH,
    scratch_types=[
        pltpu.VMEM((CHUNK_WORDS,), jnp.float32),
        pltpu.VMEM((CHUNK_WORDS,), jnp.float32),
        pltpu.VMEM((D,), jnp.float32),
        pltpu.VMEM((CHUNK_WORDS,), jnp.float32),
        pltpu.SemaphoreType.DMA,
        pltpu.SemaphoreType.DMA,
        pltpu.SemaphoreType.DMA,
    ],
)

_relu_call = pl.kernel(
    _relu_body,
    out_type=jax.ShapeDtypeStruct((NC * D,), jnp.float32),
    mesh=_MESH,
    scratch_types=[
        pltpu.VMEM((CHUNK_WORDS,), jnp.float32),
        pltpu.VMEM((CHUNK_WORDS,), jnp.float32),
        pltpu.VMEM((NW * D,), jnp.float32),
        pltpu.VMEM((NW * D,), jnp.float32),
        pltpu.VMEM((D,), jnp.float32),
        pltpu.VMEM((D,), jnp.float32),
        pltpu.VMEM((CHUNK_WORDS,), jnp.float32),
        pltpu.VMEM((NS * D,), jnp.float32),
        pltpu.VMEM((4 * D,), jnp.float32),
        pltpu.VMEM_SHARED((NS * D,), jnp.float32),
        pltpu.SemaphoreType.DMA,
        pltpu.SemaphoreType.DMA,
        pltpu.SemaphoreType.DMA,
    ],
)


@jax.jit
def kernel(x, scalars, t):
    x1 = x.reshape(-1)
    pmin, pmax, big = _minmax_call(x1)
    out2 = _relu_call(x1, pmin, pmax, t, scalars.reshape(-1), big)
    row0 = out2[:D] + out2[D:]
    return lax.dynamic_update_slice(
        big.reshape(N_ROWS + 1, D), row0.reshape(1, D), (0, 0))


# speedup vs baseline: 1.0911x; 1.0019x over previous
import jax
import jax.numpy as jnp
from jax import lax
from jax.experimental import pallas as pl
from jax.experimental.pallas import tpu as pltpu
from jax.experimental.pallas import tpu_sc as plsc

N_ROWS = 320000
D = 128
NC = 2
NS = 16
NW = NC * NS
LANES = 16
G = D // LANES
ROWS_PER_W = N_ROWS // NW
CHUNK_ROWS = 250
CHUNK_WORDS = CHUNK_ROWS * D
NCHUNK = ROWS_PER_W // CHUNK_ROWS
WORDS_PER_W = ROWS_PER_W * D
U = 5

_MESH = plsc.VectorSubcoreMesh(core_axis_name="c", subcore_axis_name="s")


def _wid():
    return lax.axis_index("c") * NS + lax.axis_index("s")


def _minmax_body(x_hbm, pmin_hbm, pmax_hbm, big_hbm,
                 buf0, buf1, stage, zbuf, sem0, sem1, zsem):
    wid = _wid()
    base = wid * WORDS_PER_W
    zbase = (wid * (ROWS_PER_W // 2) + 1) * D

    def dma(c, buf, sem):
        return pltpu.make_async_copy(
            x_hbm.at[pl.ds(base + c * CHUNK_WORDS, CHUNK_WORDS)], buf, sem)

    def zdma(c):
        return pltpu.make_async_copy(
            zbuf, big_hbm.at[pl.ds(zbase + c * CHUNK_WORDS, CHUNK_WORDS)],
            zsem)

    dma(0, buf0, sem0).start()
    dma(1, buf1, sem1).start()

    zero = jnp.zeros((LANES,), jnp.float32)

    def zinit(i, _):
        for u in range(8):
            zbuf[pl.ds((i * 8 + u) * LANES, LANES)] = zero
        return 0

    lax.fori_loop(0, CHUNK_WORDS // (8 * LANES), zinit, 0)

    inf = jnp.full((LANES,), jnp.inf, jnp.float32)
    ninf = jnp.full((LANES,), -jnp.inf, jnp.float32)
    acc0 = tuple([inf] * G) + tuple([ninf] * G)

    def chunk_compute(buf, acc):
        def row_body(i, a):
            mins = list(a[:G])
            maxs = list(a[G:])
            for u in range(U):
                roff = (i * U + u) * D
                for g in range(G):
                    v = buf[pl.ds(roff + g * LANES, LANES)]
                    mins[g] = jnp.minimum(mins[g], v)
                    maxs[g] = jnp.maximum(maxs[g], v)
            return tuple(mins) + tuple(maxs)
        return lax.fori_loop(0, CHUNK_ROWS // U, row_body, acc)

    def pair_body(p, acc):
        c = p * 2
        dma(c, buf0, sem0).wait()
        acc = chunk_compute(buf0, acc)

        @pl.when(c + 2 < NCHUNK)
        def _():
            dma(c + 2, buf0, sem0).start()

        @pl.when(p > 0)
        def _():
            zdma(p - 1).wait()

        zdma(p).start()

        dma(c + 1, buf1, sem1).wait()
        acc = chunk_compute(buf1, acc)

        @pl.when(c + 3 < NCHUNK)
        def _():
            dma(c + 3, buf1, sem1).start()

        return acc

    acc = lax.fori_loop(0, NCHUNK // 2, pair_body, acc0)
    zdma(NCHUNK // 2 - 1).wait()

    for g in range(G):
        stage[pl.ds(g * LANES, LANES)] = acc[g]
    pltpu.sync_copy(stage, pmin_hbm.at[pl.ds(wid * D, D)])
    for g in range(G):
        stage[pl.ds(g * LANES, LANES)] = acc[G + g]
    pltpu.sync_copy(stage, pmax_hbm.at[pl.ds(wid * D, D)])


def _relu_body(x_hbm, pmin_hbm, pmax_hbm, t_hbm, sc_hbm, big_hbm,
               out2_hbm,
               buf0, buf1, pm_v, px_v, t_v, stage, zbuf, pr_v, sc_v, shared,
               sem0, sem1, zsem):
    wid = _wid()
    base = wid * WORDS_PER_W
    zbase = ((N_ROWS // 2) + wid * (ROWS_PER_W // 2) + 1) * D

    def dma(c, buf, sem):
        return pltpu.make_async_copy(
            x_hbm.at[pl.ds(base + c * CHUNK_WORDS, CHUNK_WORDS)], buf, sem)

    def zdma(c):
        return pltpu.make_async_copy(
            zbuf, big_hbm.at[pl.ds(zbase + c * CHUNK_WORDS, CHUNK_WORDS)],
            zsem)

    dma(0, buf0, sem0).start()
    dma(1, buf1, sem1).start()

    zero = jnp.zeros((LANES,), jnp.float32)

    def zinit(i, _):
        for u in range(8):
            zbuf[pl.ds((i * 8 + u) * LANES, LANES)] = zero
        return 0

    lax.fori_loop(0, CHUNK_WORDS // (8 * LANES), zinit, 0)

    pltpu.sync_copy(pmin_hbm, pm_v)
    pltpu.sync_copy(pmax_hbm, px_v)
    pltpu.sync_copy(t_hbm, t_v)

    inf = jnp.full((LANES,), jnp.inf, jnp.float32)
    ninf = jnp.full((LANES,), -jnp.inf, jnp.float32)

    def red_body(w, a):
        mins = list(a[:G])
        maxs = list(a[G:])
        for g in range(G):
            mins[g] = jnp.minimum(mins[g], pm_v[pl.ds(w * D + g * LANES, LANES)])
            maxs[g] = jnp.maximum(maxs[g], px_v[pl.ds(w * D + g * LANES, LANES)])
        return tuple(mins) + tuple(maxs)

    red = lax.fori_loop(0, NW, red_body, tuple([inf] * G) + tuple([ninf] * G))
    bias = []
    for g in range(G):
        tg = t_v[pl.ds(g * LANES, LANES)]
        bias.append(tg * red[G + g] + (1.0 - tg) * red[g])
    bias = tuple(bias)

    acc0 = tuple([zero] * G)

    cbias = tuple(jnp.float32(CHUNK_ROWS) * bias[g] for g in range(G))

    def chunk_compute(buf, acc):
        def row_body(i, a):
            sums = list(a)
            for u in range(U):
                roff = (i * U + u) * D
                for g in range(G):
                    v = buf[pl.ds(roff + g * LANES, LANES)]
                    sums[g] = sums[g] + jnp.maximum(v, bias[g])
            return tuple(sums)
        csum = lax.fori_loop(0, CHUNK_ROWS // U, row_body, tuple([zero] * G))
        return tuple(acc[g] + (csum[g] - cbias[g]) for g in range(G))

    def pair_body(p, acc):
        c = p * 2
        dma(c, buf0, sem0).wait()
        acc = chunk_compute(buf0, acc)

        @pl.when(c + 2 < NCHUNK)
        def _():
            dma(c + 2, buf0, sem0).start()

        @pl.when(p > 0)
        def _():
            zdma(p - 1).wait()

        zdma(p).start()

        dma(c + 1, buf1, sem1).wait()
        acc = chunk_compute(buf1, acc)

        @pl.when(c + 3 < NCHUNK)
        def _():
            dma(c + 3, buf1, sem1).start()

        return acc

    acc = lax.fori_loop(0, NCHUNK // 2, pair_body, acc0)
    zdma(NCHUNK // 2 - 1).wait()

    sid = lax.axis_index("s")
    for g in range(G):
        stage[pl.ds(g * LANES, LANES)] = acc[g]
    pltpu.sync_copy(stage, shared.at[pl.ds(sid * D, D)])

    plsc.subcore_barrier()

    cid = lax.axis_index("c")

    @pl.when(sid == 0)
    def _():
        pltpu.sync_copy(shared, pr_v)
        pltpu.sync_copy(sc_hbm, sc_v)

        def sum_body(w, a):
            return tuple(a[g] + pr_v[pl.ds(w * D + g * LANES, LANES)]
                         for g in range(G))

        ssum = lax.fori_loop(0, NS, sum_body, tuple([zero] * G))
        cnt = jnp.float32(N_ROWS)
        on_core0 = (cid == 0).astype(jnp.float32)
        for g in range(G):
            s0 = sc_v[pl.ds(0 * D + g * LANES, LANES)]
            s1 = sc_v[pl.ds(1 * D + g * LANES, LANES)]
            s2 = sc_v[pl.ds(2 * D + g * LANES, LANES)]
            s3 = sc_v[pl.ds(3 * D + g * LANES, LANES)]
            row = (s3 * ssum[g]
                   + on_core0 * (s0 * cnt + s1 * red[g] + s2 * red[G + g]))
            stage[pl.ds(g * LANES, LANES)] = row
        pltpu.sync_copy(stage, out2_hbm.at[pl.ds(cid * D, D)])


_minmax_call = pl.kernel(
    _minmax_body,
    out_type=(jax.ShapeDtypeStruct((NW * D,), jnp.float32),
              jax.ShapeDtypeStruct((NW * D,), jnp.float32),
              jax.ShapeDtypeStruct(((N_ROWS + 1) * D,), jnp.float32)),
    mesh=_MESH,
    scratch_types=[
        pltpu.VMEM((CHUNK_WORDS,), jnp.float32),
        pltpu.VMEM((CHUNK_WORDS,), jnp.float32),
        pltpu.VMEM((D,), jnp.float32),
        pltpu.VMEM((CHUNK_WORDS,), jnp.float32),
        pltpu.SemaphoreType.DMA,
        pltpu.SemaphoreType.DMA,
        pltpu.SemaphoreType.DMA,
    ],
)

_relu_call = pl.kernel(
    _relu_body,
    out_type=jax.ShapeDtypeStruct((NC * D,), jnp.float32),
    mesh=_MESH,
    scratch_types=[
        pltpu.VMEM((CHUNK_WORDS,), jnp.float32),
        pltpu.VMEM((CHUNK_WORDS,), jnp.float32),
        pltpu.VMEM((NW * D,), jnp.float32),
        pltpu.VMEM((NW * D,), jnp.float32),
        pltpu.VMEM((D,), jnp.float32),
        pltpu.VMEM((D,), jnp.float32),
        pltpu.VMEM((CHUNK_WORDS,), jnp.float32),
        pltpu.VMEM((NS * D,), jnp.float32),
        pltpu.VMEM((4 * D,), jnp.float32),
        pltpu.VMEM_SHARED((NS * D,), jnp.float32),
        pltpu.SemaphoreType.DMA,
        pltpu.SemaphoreType.DMA,
        pltpu.SemaphoreType.DMA,
    ],
)


@jax.jit
def kernel(x, scalars, t):
    x1 = x.reshape(-1)
    pmin, pmax, big = _minmax_call(x1)
    out2 = _relu_call(x1, pmin, pmax, t, scalars.reshape(-1), big)
    row0 = out2[:D] + out2[D:]
    return lax.dynamic_update_slice(
        big.reshape(N_ROWS + 1, D), row0.reshape(1, D), (0, 0))
